# baseline (speedup 1.0000x reference)
import functools

import jax
import jax.numpy as jnp
from jax.experimental import pallas as pl
from jax.experimental.pallas import tpu as pltpu

_TOP_K = 8
_TEMP_INV = 1.0 / 0.3


def _router_body(x_ref, w_ref, idx_ref, score_ref, me_ref, ce_ref, acc_ref):
    j = pl.program_id(1)
    nj = pl.num_programs(1)
    x = x_ref[...]
    w = w_ref[...]
    partial = jax.lax.dot_general(
        x, w, (((1,), (1,)), ((), ())), preferred_element_type=jnp.float32
    )

    @pl.when(j == 0)
    def _first():
        acc_ref[...] = partial

    @pl.when(j != 0)
    def _accum():
        acc_ref[...] += partial

    @pl.when(j == nj - 1)
    def _epilogue():
        _gate_epilogue(acc_ref[...], idx_ref, score_ref, me_ref, ce_ref)


def _gate_epilogue(logits, idx_ref, score_ref, me_ref, ce_ref):
    n_expert = logits.shape[1]
    lt = logits.T
    row = jax.lax.broadcasted_iota(jnp.int32, lt.shape, 0)

    lx = lt
    vals = []
    idxs = []
    ce_mask = None
    for k in range(_TOP_K):
        m = jnp.max(lx, axis=0, keepdims=True)
        am = jnp.min(jnp.where(lx == m, row, n_expert), axis=0,
                     keepdims=True)
        vals.append(m)
        idxs.append(am)
        hit = row == am
        if k == 0:
            ce_mask = hit
        lx = jnp.where(hit, -jnp.inf, lx)
    topv = jnp.concatenate(vals, axis=0)
    topi = jnp.concatenate(idxs, axis=0)

    e = jnp.exp(topv - vals[0])
    score_ref[...] = (e / jnp.sum(e, axis=0, keepdims=True)).T
    idx_ref[...] = topi.T

    t = jnp.exp((lt - vals[0]) * _TEMP_INV)
    p = t / jnp.sum(t, axis=0, keepdims=True)
    me_ref[0, ...] = jnp.sum(p, axis=1, keepdims=True).T
    ce_ref[0, ...] = jnp.sum(ce_mask.astype(jnp.float32), axis=1,
                             keepdims=True).T


def _loss_body(me_ref, ce_ref, loss_ref, *, n_tokens, n_expert):
    me = jnp.sum(me_ref[:, 0, :], axis=0, keepdims=True)
    ce = jnp.sum(ce_ref[:, 0, :], axis=0, keepdims=True)
    hot_value = n_expert / n_tokens
    loss_ref[...] = jnp.sum(me * ce, axis=1, keepdims=True) * (
        hot_value / n_tokens)


def kernel(inp, W):
    n_tokens, d_model = inp.shape
    n_expert = W.shape[0]
    tile = 1024
    while n_tokens % tile:
        tile //= 2
    grid = n_tokens // tile
    ksplit = 2
    dchunk = d_model // ksplit

    idx, score, me_parts, ce_parts = pl.pallas_call(
        _router_body,
        grid=(grid, ksplit),
        in_specs=[
            pl.BlockSpec((tile, dchunk), lambda i, j: (i, j)),
            pl.BlockSpec((n_expert, dchunk), lambda i, j: (0, j)),
        ],
        out_specs=[
            pl.BlockSpec((tile, _TOP_K), lambda i, j: (i, 0)),
            pl.BlockSpec((tile, _TOP_K), lambda i, j: (i, 0)),
            pl.BlockSpec((1, 1, n_expert), lambda i, j: (i, 0, 0)),
            pl.BlockSpec((1, 1, n_expert), lambda i, j: (i, 0, 0)),
        ],
        out_shape=[
            jax.ShapeDtypeStruct((n_tokens, _TOP_K), jnp.int32),
            jax.ShapeDtypeStruct((n_tokens, _TOP_K), jnp.float32),
            jax.ShapeDtypeStruct((grid, 1, n_expert), jnp.float32),
            jax.ShapeDtypeStruct((grid, 1, n_expert), jnp.float32),
        ],
        scratch_shapes=[
            pltpu.VMEM((tile, n_expert), jnp.float32),
        ],
        compiler_params=pltpu.CompilerParams(
            dimension_semantics=("parallel", "arbitrary"),
        ),
    )(inp, W)

    loss = pl.pallas_call(
        functools.partial(_loss_body, n_tokens=n_tokens, n_expert=n_expert),
        out_shape=jax.ShapeDtypeStruct((1, 1), jnp.float32),
    )(me_parts, ce_parts)
    return idx, score, loss.reshape(())


# speedup vs baseline: 1.1869x; 1.1869x over previous
import functools

import jax
import jax.numpy as jnp
from jax.experimental import pallas as pl
from jax.experimental.pallas import tpu as pltpu

_TOP_K = 8
_TEMP_INV = 1.0 / 0.3


def _router_body(x_ref, w_ref, idx_ref, score_ref, me_ref, ce_ref):
    x = x_ref[...]
    w = w_ref[...]
    logits = jax.lax.dot_general(
        x, w, (((1,), (1,)), ((), ())), preferred_element_type=jnp.float32
    )
    n_expert = logits.shape[1]
    lt = logits.T
    row = jax.lax.broadcasted_iota(jnp.int32, lt.shape, 0)

    lx = lt
    vals = []
    idxs = []
    ce_mask = None
    for k in range(_TOP_K):
        m = jnp.max(lx, axis=0, keepdims=True)
        am = jnp.min(jnp.where(lx == m, row, n_expert), axis=0,
                     keepdims=True)
        vals.append(m)
        idxs.append(am)
        hit = row == am
        if k == 0:
            ce_mask = hit
        lx = jnp.where(hit, -jnp.inf, lx)
    topv = jnp.concatenate(vals, axis=0)
    topi = jnp.concatenate(idxs, axis=0)

    e = jnp.exp(topv - vals[0])
    score_ref[...] = (e / jnp.sum(e, axis=0, keepdims=True)).T
    idx_ref[...] = topi.T

    t = jnp.exp((lt - vals[0]) * _TEMP_INV)
    p = t / jnp.sum(t, axis=0, keepdims=True)
    me_ref[0, ...] = jnp.sum(p, axis=1, keepdims=True).T
    ce_ref[0, ...] = jnp.sum(ce_mask.astype(jnp.float32), axis=1,
                             keepdims=True).T


def _loss_body(me_ref, ce_ref, loss_ref, *, n_tokens, n_expert):
    me = jnp.sum(me_ref[:, 0, :], axis=0, keepdims=True)
    ce = jnp.sum(ce_ref[:, 0, :], axis=0, keepdims=True)
    hot_value = n_expert / n_tokens
    loss_ref[...] = jnp.sum(me * ce, axis=1, keepdims=True) * (
        hot_value / n_tokens)


def kernel(inp, W):
    n_tokens, d_model = inp.shape
    n_expert = W.shape[0]
    tile = 1024
    while n_tokens % tile:
        tile //= 2
    grid = n_tokens // tile

    idx, score, me_parts, ce_parts = pl.pallas_call(
        _router_body,
        grid=(grid,),
        in_specs=[
            pl.BlockSpec((tile, d_model), lambda i: (i, 0)),
            pl.BlockSpec((n_expert, d_model), lambda i: (0, 0)),
        ],
        out_specs=[
            pl.BlockSpec((tile, _TOP_K), lambda i: (i, 0)),
            pl.BlockSpec((tile, _TOP_K), lambda i: (i, 0)),
            pl.BlockSpec((1, 1, n_expert), lambda i: (i, 0, 0)),
            pl.BlockSpec((1, 1, n_expert), lambda i: (i, 0, 0)),
        ],
        out_shape=[
            jax.ShapeDtypeStruct((n_tokens, _TOP_K), jnp.int32),
            jax.ShapeDtypeStruct((n_tokens, _TOP_K), jnp.float32),
            jax.ShapeDtypeStruct((grid, 1, n_expert), jnp.float32),
            jax.ShapeDtypeStruct((grid, 1, n_expert), jnp.float32),
        ],
        compiler_params=pltpu.CompilerParams(
            dimension_semantics=("parallel",),
        ),
    )(inp, W)

    loss = pl.pallas_call(
        functools.partial(_loss_body, n_tokens=n_tokens, n_expert=n_expert),
        out_shape=jax.ShapeDtypeStruct((1, 1), jnp.float32),
    )(me_parts, ce_parts)
    return idx, score, loss.reshape(())
